# Initial kernel scaffold; baseline (speedup 1.0000x reference)
import jax
import jax.numpy as jnp
from jax.experimental import pallas as pl


def kernel(x, edge_index, batch, emb, Ws, bs, W_out, b_out):
    raise NotImplementedError("write your pallas kernel here")


# speedup vs baseline: 35.1375x; 35.1375x over previous
import functools

import jax
import jax.numpy as jnp
from jax import lax
from jax.experimental import pallas as pl
from jax.experimental.pallas import tpu as pltpu
from jax.experimental.pallas import tpu_sc as plsc

N = 100000
E = 3200000
IN_DIM = 128
HID = 16
OUT_DIM = 10
NUM_LAYERS = 15
NUM_GRAPHS = 64

NP = 100352
NPR = NP // 8
NCORES = 2
NSUB = 16
NW = NCORES * NSUB
CH = 128
EP = 32 * 100352
ERP = EP // CH
EROWS = ERP // NW
BLK = 16
NBLK = EROWS // BLK
NPT = NP // NSUB
ZR = 784

XR = NP // CH
XU = XR // 8
XU_BASE = XU // NW
XU_REM = XU % NW


def _fill_rows(ref, nrows, value):
  v = jnp.full((HID,), value, jnp.float32)

  def body(i, _):
    ref[i] = v
    return 0

  lax.fori_loop(0, nrows, body, 0)


def _worker_id():
  return lax.axis_index("c") * NSUB + lax.axis_index("s")


def _zero_acc(zbuf, acc_sh, sid):
  _fill_rows(zbuf, ZR, 0.0)
  for k in range(NPT // ZR):
    pltpu.sync_copy(zbuf, acc_sh.at[pl.ds(sid * NPT + k * ZR, ZR)])


def _writeback(acc_sh, out_h, cid, sid):
  pltpu.sync_copy(
      acc_sh.at[pl.ds(sid * NPT, NPT)],
      out_h.at[cid, pl.ds(sid * NPT, NPT)],
  )


def _agg_body(src_h, dst_h, g_h, out_h, idx_s, idx_d,
              rows0, rows1, zbuf, acc_sh, sem0, sem1):
  cid = lax.axis_index("c")
  sid = lax.axis_index("s")
  wid = _worker_id()
  _zero_acc(zbuf, acc_sh, sid)
  plsc.subcore_barrier()

  row0 = wid * EROWS

  def block(b, _):
    base = row0 + b * BLK
    pltpu.sync_copy(src_h.at[pl.ds(base, BLK)], idx_s)
    pltpu.sync_copy(dst_h.at[pl.ds(base, BLK)], idx_d)
    cp = pltpu.async_copy(g_h.at[idx_s.at[0]], rows0, sem0)
    for j in range(1, BLK):
      rows_n = rows1 if j % 2 else rows0
      rows_p = rows0 if j % 2 else rows1
      sem_n = sem1 if j % 2 else sem0
      cpn = pltpu.async_copy(g_h.at[idx_s.at[j]], rows_n, sem_n)
      cp.wait()
      pltpu.sync_copy(rows_p, acc_sh.at[idx_d.at[j - 1]], add=True)
      cp = cpn
    cp.wait()
    last = rows1 if (BLK - 1) % 2 else rows0
    pltpu.sync_copy(last, acc_sh.at[idx_d.at[BLK - 1]], add=True)
    return 0

  lax.fori_loop(0, NBLK, block, 0)

  plsc.subcore_barrier()
  _writeback(acc_sh, out_h, cid, sid)


def _sc_agg(src2, dst2, g):
  mesh = plsc.VectorSubcoreMesh(core_axis_name="c", subcore_axis_name="s")
  f = pl.kernel(
      _agg_body,
      out_type=jax.ShapeDtypeStruct((NCORES, NP, HID), jnp.float32),
      mesh=mesh,
      scratch_types=[
          pltpu.VMEM((BLK, CH), jnp.int32),
          pltpu.VMEM((BLK, CH), jnp.int32),
          pltpu.VMEM((CH, HID), jnp.float32),
          pltpu.VMEM((CH, HID), jnp.float32),
          pltpu.VMEM((ZR, HID), jnp.float32),
          pltpu.VMEM_SHARED((NP, HID), jnp.float32),
          pltpu.SemaphoreType.DMA,
          pltpu.SemaphoreType.DMA,
      ],
      compiler_params=pltpu.CompilerParams(use_tc_tiling_on_sc=False),
  )
  return f(src2, dst2, g)


def _emb_body(x_h, emb2_h, s_h, out_h, idx_x, rows, svals):
  wid = _worker_id()
  u0 = wid * XU_BASE + jnp.minimum(wid, XU_REM)
  nu = XU_BASE + jnp.where(wid < XU_REM, 1, 0)

  def body(u, _):
    unit = u0 + u
    pltpu.sync_copy(x_h.at[pl.ds(unit * 8, 8)], idx_x)
    for r in range(8):
      row = unit * 8 + r
      pltpu.sync_copy(emb2_h.at[idx_x.at[r]], rows)
      pltpu.sync_copy(s_h.at[pl.ds(row * CH, CH)], svals)

      def mul(i, _):
        rows[i] = rows[i] * svals[i]
        return 0

      lax.fori_loop(0, CH, mul, 0)
      pltpu.sync_copy(rows, out_h.at[pl.ds(row * CH, CH)])
    return 0

  lax.fori_loop(0, nu, body, 0)


def _sc_emb(x2, emb2, s16):
  mesh = plsc.VectorSubcoreMesh(core_axis_name="c", subcore_axis_name="s")
  f = pl.kernel(
      _emb_body,
      out_type=jax.ShapeDtypeStruct((NP, HID), jnp.float32),
      mesh=mesh,
      scratch_types=[
          pltpu.VMEM((8, CH), jnp.int32),
          pltpu.VMEM((CH, HID), jnp.float32),
          pltpu.VMEM((CH, HID), jnp.float32),
      ],
      compiler_params=pltpu.CompilerParams(use_tc_tiling_on_sc=False),
  )
  return f(x2, emb2, s16)


def _prep(deg, emb, w0):
  rb = 1568
  grid = NPR // rb

  def body(deg_ref, emb_ref, w0_ref, s_ref, emb2_ref):
    d = deg_ref[0] + deg_ref[1] + 1.0
    s_ref[...] = lax.rsqrt(d)

    @pl.when(pl.program_id(0) == 0)
    def _():
      emb2_ref[...] = jnp.dot(emb_ref[...], w0_ref[...],
                              preferred_element_type=jnp.float32,
                              precision=lax.Precision.HIGHEST)

  return pl.pallas_call(
      body,
      grid=(grid,),
      in_specs=[
          pl.BlockSpec((2, rb, 128), lambda i: (0, i, 0)),
          pl.BlockSpec((IN_DIM, HID), lambda i: (0, 0)),
          pl.BlockSpec((HID, HID), lambda i: (0, 0)),
      ],
      out_specs=[
          pl.BlockSpec((rb, 128), lambda i: (i, 0)),
          pl.BlockSpec((IN_DIM, HID), lambda i: (0, 0)),
      ],
      out_shape=[
          jax.ShapeDtypeStruct((NPR, 128), jnp.float32),
          jax.ShapeDtypeStruct((IN_DIM, HID), jnp.float32),
      ],
  )(deg, emb, w0)


def _dense(acc, g, s16, w_next, b_i):
  rb = 1568
  grid = NPR // rb

  def body(acc_ref, g_ref, s_ref, w_ref, b_ref, out_ref):
    a = acc_ref[0] + acc_ref[1] + g_ref[...]
    b128 = jnp.reshape(
        jnp.broadcast_to(jnp.reshape(b_ref[...], (1, 1, HID)), (1, 8, HID)),
        (1, 128))
    h = s_ref[...] * a + b128
    u = jnp.maximum(h, 0.0)
    wb = jnp.reshape(
        jnp.broadcast_to(jnp.reshape(w_ref[...], (1, HID, 1, HID)),
                         (8, HID, 8, HID)), (128, 128))
    ii = lax.broadcasted_iota(jnp.int32, (128, 128), 0) // HID
    jj = lax.broadcasted_iota(jnp.int32, (128, 128), 1) // HID
    w128 = jnp.where(ii == jj, wb, 0.0)
    out_ref[...] = s_ref[...] * jnp.dot(u, w128,
                                        preferred_element_type=jnp.float32,
                                        precision=lax.Precision.HIGHEST)

  return pl.pallas_call(
      body,
      grid=(grid,),
      in_specs=[
          pl.BlockSpec((2, rb, 128), lambda i: (0, i, 0)),
          pl.BlockSpec((rb, 128), lambda i: (i, 0)),
          pl.BlockSpec((rb, 128), lambda i: (i, 0)),
          pl.BlockSpec((HID, HID), lambda i: (0, 0)),
          pl.BlockSpec((1, HID), lambda i: (0, 0)),
      ],
      out_specs=pl.BlockSpec((rb, 128), lambda i: (i, 0)),
      out_shape=jax.ShapeDtypeStruct((NPR, 128), jnp.float32),
  )(acc, g, s16, w_next, b_i)


def _final(acc, g, s16v, b14, batch3, w_out, b_out):
  bsz = 2048
  grid = NP // bsz

  def body(acc_ref, g_ref, s_ref, b_ref, batch_ref, wout_ref, bout_ref,
           out_ref, pool_acc, cnt_acc):
    i = pl.program_id(0)

    @pl.when(i == 0)
    def _():
      pool_acc[...] = jnp.zeros_like(pool_acc)
      cnt_acc[...] = jnp.zeros_like(cnt_acc)

    a = acc_ref[0] + acc_ref[1] + g_ref[...]
    h = s_ref[...] * a + b_ref[...]
    bt = batch_ref[0]
    gids = lax.broadcasted_iota(jnp.int32, (NUM_GRAPHS, bsz), 0)
    vidx = lax.broadcasted_iota(jnp.int32, (NUM_GRAPHS, bsz), 1) + i * bsz
    oh = jnp.where((bt == gids) & (vidx < N), 1.0, 0.0)
    pool_acc[...] += jnp.dot(oh, h, preferred_element_type=jnp.float32,
                             precision=lax.Precision.HIGHEST)
    cnt_acc[...] += jnp.sum(oh, axis=1, keepdims=True)

    @pl.when(i == grid - 1)
    def _():
      pooled = pool_acc[...] / jnp.maximum(cnt_acc[...], 1.0)
      out_ref[...] = jnp.dot(pooled, wout_ref[...],
                             preferred_element_type=jnp.float32,
                             precision=lax.Precision.HIGHEST) + bout_ref[...]

  return pl.pallas_call(
      body,
      grid=(grid,),
      in_specs=[
          pl.BlockSpec((2, bsz, HID), lambda i: (0, i, 0)),
          pl.BlockSpec((bsz, HID), lambda i: (i, 0)),
          pl.BlockSpec((bsz, HID), lambda i: (i, 0)),
          pl.BlockSpec((1, HID), lambda i: (0, 0)),
          pl.BlockSpec((1, 1, bsz), lambda i: (i, 0, 0)),
          pl.BlockSpec((HID, OUT_DIM), lambda i: (0, 0)),
          pl.BlockSpec((1, OUT_DIM), lambda i: (0, 0)),
      ],
      out_specs=pl.BlockSpec((NUM_GRAPHS, OUT_DIM), lambda i: (0, 0)),
      out_shape=jax.ShapeDtypeStruct((NUM_GRAPHS, OUT_DIM), jnp.float32),
      scratch_shapes=[
          pltpu.VMEM((NUM_GRAPHS, HID), jnp.float32),
          pltpu.VMEM((NUM_GRAPHS, 1), jnp.float32),
      ],
  )(acc, g, s16v, b14, batch3, w_out, b_out)


def kernel(x, edge_index, batch, emb, Ws, bs, W_out, b_out):
  pad = jnp.arange(EP - E, dtype=jnp.int32) % (NP - N) + N
  src2 = jnp.concatenate(
      [edge_index[0].astype(jnp.int32), pad]).reshape(ERP, CH)
  dst2 = jnp.concatenate(
      [edge_index[1].astype(jnp.int32), pad]).reshape(ERP, CH)
  xp = jnp.concatenate(
      [x.astype(jnp.int32), jnp.zeros((NP - N,), jnp.int32)]).reshape(XR, CH)
  batch3 = jnp.concatenate(
      [batch.astype(jnp.int32),
       jnp.full((NP - N,), NUM_GRAPHS - 1, jnp.int32)]).reshape(NP // 2048, 1,
                                                                2048)

  deg = _sc_agg(src2, dst2, jnp.ones((NP, HID), jnp.float32))
  s16, emb2 = _prep(deg.reshape(2, NPR, 128), emb, Ws[0])
  s16v = s16.reshape(NP, HID)
  g = _sc_emb(xp, emb2, s16v)

  for i in range(NUM_LAYERS - 1):
    acc = _sc_agg(src2, dst2, g)
    g = _dense(acc.reshape(2, NPR, 128), g.reshape(NPR, 128), s16,
               Ws[i + 1], bs[i].reshape(1, HID)).reshape(NP, HID)

  acc = _sc_agg(src2, dst2, g)
  out = _final(acc, g, s16v, bs[NUM_LAYERS - 1].reshape(1, HID), batch3,
               W_out, b_out.reshape(1, OUT_DIM))
  return out
